# baseline (speedup 1.0000x reference)
import functools
import jax
import jax.numpy as jnp
from jax import lax
from jax.experimental import pallas as pl
from jax.experimental.pallas import tpu as pltpu
from jax.experimental.pallas import tpu_sc as plsc

B = 128
V = 100000
F = 800
SUB = 8
C = F * SUB
K = (V + C - 1) // C
TAIL = V - (K - 1) * C
NFV = V // F
NW = 32
RPW = B // NW
NV16 = F // 16

NEG_BIG = -3.0e38


def _lane_shift_right(x, sh):
    r, w = x.shape
    return jnp.concatenate(
        [jnp.zeros((r, sh), x.dtype), x[:, :w - sh]], axis=1)


def _lane_cumsum(x):
    w = x.shape[1]
    sh = 1
    while sh < w:
        x = x + _lane_shift_right(x, sh)
        sh *= 2
    return x


def _g16(x, idx):
    return jax.lax.gather(
        x, idx[:, None],
        jax.lax.GatherDimensionNumbers(
            offset_dims=(), collapsed_slice_dims=(0,),
            start_index_map=(0,)),
        (1,), mode=jax.lax.GatherScatterMode.PROMISE_IN_BOUNDS)


def _substats(scaled, masked):
    mks, sks = [], []
    for i in range(SUB):
        xs = scaled[:, i * F:(i + 1) * F]
        mk = jnp.max(xs, axis=1, keepdims=True)
        e = jnp.exp(xs - mk)
        if masked:
            lane = jax.lax.broadcasted_iota(jnp.int32, (B, F), 1)
            e = jnp.where(i * F + lane < TAIL, e, 0.0)
        sks.append(jnp.sum(e, axis=1, keepdims=True))
        mks.append(mk)
    return jnp.concatenate(mks, axis=1), jnp.concatenate(sks, axis=1)


def _stats_kernel(logits_ref, invt_ref, u_ref, gidx_ref, scal_ref,
                  m3, s3):
    k = pl.program_id(0)
    x = logits_ref[...]
    invt = invt_ref[...]

    @pl.when(k < K - 1)
    def _full_block():
        mk, sk = _substats(x * invt, masked=False)
        m3[pl.ds(k, 1)] = mk.reshape(1, B, SUB)
        s3[pl.ds(k, 1)] = sk.reshape(1, B, SUB)

    @pl.when(k == K - 1)
    def _tail_and_merge():
        lane = jax.lax.broadcasted_iota(jnp.int32, (B, C), 1)
        scaled = jnp.where(lane < TAIL, x * invt, NEG_BIG)
        mk, sk = _substats(scaled, masked=True)
        m3[pl.ds(k, 1)] = mk.reshape(1, B, SUB)
        s3[pl.ds(k, 1)] = sk.reshape(1, B, SUB)

        m = jnp.full((B, 1), NEG_BIG, jnp.float32)
        for kk in range(K):
            m = jnp.maximum(m, jnp.max(m3[kk], axis=1, keepdims=True))
        z = jnp.zeros((B, 1), jnp.float32)
        for kk in range(K):
            a = s3[kk] * jnp.exp(m3[kk] - m)
            z = z + jnp.sum(a, axis=1, keepdims=True)
        t = u_ref[...] * z
        run = jnp.zeros((B, 1), jnp.float32)
        cnt = jnp.zeros((B, 1), jnp.float32)
        pexc = jnp.zeros((B, 1), jnp.float32)
        for kk in range(K):
            a = s3[kk] * jnp.exp(m3[kk] - m)
            p = run + _lane_cumsum(a)
            below = p < t
            cnt = cnt + jnp.sum(
                jnp.where(below, 1.0, 0.0), axis=1, keepdims=True)
            pexc = pexc + jnp.sum(
                jnp.where(below, a, 0.0), axis=1, keepdims=True)
            run = run + jnp.sum(a, axis=1, keepdims=True)
        kst = jnp.minimum(cnt, float(NFV - 1))
        ridx = jax.lax.broadcasted_iota(jnp.int32, (B, 1), 0).astype(
            jnp.float32)
        ridx = jax.lax.broadcasted_iota(jnp.int32, (B, 1), 0).astype(
            jnp.float32)
        gidx_ref[...] = (ridx * float(NFV) + kst).astype(jnp.int32)
        scal_ref[...] = jnp.concatenate(
            [m, t, pexc, kst, jnp.zeros((B, 4), jnp.float32)], axis=1)


def _sc_pick(table_hbm, gidx_hbm, scalv_hbm, out_hbm,
             idx_v, rows_v, scal_v, outv_v, sem):
    wid = lax.axis_index("s") * 2 + lax.axis_index("c")
    pltpu.sync_copy(gidx_hbm.at[wid], idx_v)
    pltpu.async_copy(table_hbm.at[idx_v], rows_v, sem).wait()
    lane = jax.lax.broadcasted_iota(jnp.int32, (16,), 0)
    last = jnp.full((16,), 15, jnp.int32)
    outvec = jnp.zeros((16,), jnp.int32)
    for j in range(RPW):
        pltpu.sync_copy(scalv_hbm.at[wid * RPW + j], scal_v)
        m_v = scal_v[0]
        t_v = scal_v[1]
        pexc_v = scal_v[2]
        kst_v = scal_v[3]
        invt_v = scal_v[4]

        def body(i, carry):
            off_v, cnt_v = carry
            v = rows_v[j, pl.ds(i * 16, 16)]
            e = jnp.exp(v * invt_v - m_v)
            lane16 = jax.lax.broadcasted_iota(jnp.int32, (16,), 0)
            for _sh in (1, 2, 4, 8):
                g = _g16(e, jnp.maximum(lane16 - _sh, 0))
                e = e + jnp.where(lane16 >= _sh, g, 0.0)
            p = off_v + e
            off_v = _g16(p, jnp.full((16,), 15, jnp.int32))
            below = p < t_v
            cnt_v = cnt_v + jnp.where(below, 1.0, 0.0)
            return off_v, cnt_v

        _, cnt_v = lax.fori_loop(
            0, NV16, body, (pexc_v, jnp.zeros((16,), jnp.float32)))
        for _sh in (1, 2, 4, 8):
            g = _g16(cnt_v, jnp.maximum(lane - _sh, 0))
            cnt_v = cnt_v + jnp.where(lane >= _sh, g, 0.0)
        tot = _g16(cnt_v, jnp.full((16,), 15, jnp.int32))
        samp = jnp.minimum(
            kst_v * float(F) + tot, float(V - 1)).astype(jnp.int32)
        outvec = jnp.where(lane == j, samp, outvec)
    outv_v[...] = outvec
    pltpu.sync_copy(outv_v, out_hbm.at[wid])


def kernel(logits, temperatures):
    u = jax.random.uniform(jax.random.key(42), (B, 1), dtype=jnp.float32)
    invt = (1.0 / temperatures).reshape(B, 1)

    gidx, scal = pl.pallas_call(
        _stats_kernel,
        grid=(K,),
        in_specs=[
            pl.BlockSpec((B, C), lambda k: (0, k)),
            pl.BlockSpec((B, 1), lambda k: (0, 0)),
            pl.BlockSpec((B, 1), lambda k: (0, 0)),
        ],
        out_specs=[
            pl.BlockSpec((B, 1), lambda k: (0, 0)),
            pl.BlockSpec((B, 8), lambda k: (0, 0)),
        ],
        out_shape=[
            jax.ShapeDtypeStruct((B, 1), jnp.int32),
            jax.ShapeDtypeStruct((B, 8), jnp.float32),
        ],
        scratch_shapes=[
            pltpu.VMEM((K, B, SUB), jnp.float32),
            pltpu.VMEM((K, B, SUB), jnp.float32),
        ],
    )(logits, invt, u)

    gidx32 = jnp.pad(gidx.reshape(NW, RPW), ((0, 0), (0, 8 - RPW)))
    scal5 = jnp.concatenate([scal[:, :4], invt], axis=1)
    scalv = jnp.broadcast_to(
        jnp.pad(scal5, ((0, 0), (0, 3)))[:, :, None], (B, 8, 16))

    mesh = plsc.VectorSubcoreMesh(core_axis_name="c", subcore_axis_name="s")
    sc = functools.partial(
        pl.kernel, mesh=mesh,
        compiler_params=pltpu.CompilerParams(use_tc_tiling_on_sc=False),
        out_type=jax.ShapeDtypeStruct((NW, 16), jnp.int32),
        scratch_types=[
            pltpu.VMEM((8,), jnp.int32),
            pltpu.VMEM((8, F), jnp.float32),
            pltpu.VMEM((8, 16), jnp.float32),
            pltpu.VMEM((16,), jnp.int32),
            pltpu.SemaphoreType.DMA,
        ])(_sc_pick)
    table = logits.reshape(B * NFV, F)
    out32 = sc(table, gidx32, scalv)

    return out32[:, :RPW].reshape(B)
